# speedup vs baseline: 2.4034x; 2.4034x over previous
import jax
import jax.numpy as jnp
from jax.experimental import pallas as pl
from jax.experimental.pallas import tpu as pltpu

_ROWS = 512


def _body(ids_ref, w_ref, f_ref, o_ref):
    w = w_ref[...]
    w0 = w[0:1, :]
    diff = w[1:2, :] - w0
    idf = ids_ref[...]
    o_ref[...] = f_ref[...] + w0 + idf * diff


def kernel(frames_actions, token_type_ids, emb_weight):
    B, S, D = frames_actions.shape
    N = B * S
    f2 = frames_actions.reshape(N, D)
    ids = jnp.clip(token_type_ids.reshape(N, 1), 0, 1).astype(jnp.float32)

    grid = (N // _ROWS,)
    out = pl.pallas_call(
        _body,
        grid=grid,
        in_specs=[
            pl.BlockSpec((_ROWS, 1), lambda i: (i, 0)),
            pl.BlockSpec((2, D), lambda i: (0, 0)),
            pl.BlockSpec((_ROWS, D), lambda i: (i, 0)),
        ],
        out_specs=pl.BlockSpec((_ROWS, D), lambda i: (i, 0)),
        out_shape=jax.ShapeDtypeStruct((N, D), jnp.float32),
    )(ids, emb_weight, f2)
    return out.reshape(B, S, D)
